# baseline (speedup 1.0000x reference)
import functools

import jax
import jax.numpy as jnp
from jax import lax
from jax.experimental import pallas as pl
from jax.experimental.pallas import tpu as pltpu
from jax.experimental.pallas import tpu_sc as plsc

IN_CHANNELS = 128
HIDDEN = 64
N_NODES = 10000
N_EDGES = 320000

NC = 2
NS = 16
LANES = 16
NW = NC * NS
EDGES_PER_W = N_EDGES // NW
CHUNK = 400
N_CHUNKS = EDGES_PER_W // CHUNK
GROUPS = CHUNK // LANES
SUB_SPLITS = ((0, 128), (128, 128), (256, 128), (384, 16))


def _project_body(e_ref, w1a_ref, w1b_ref, b1_ref, ta_ref, tb_ref):
    e = e_ref[...]
    dn = (((1,), (0,)), ((), ()))
    ta_ref[...] = lax.dot_general(
        e, w1a_ref[...], dn, precision=lax.Precision.HIGHEST,
        preferred_element_type=jnp.float32) + b1_ref[...]
    tb_ref[...] = lax.dot_general(
        e, w1b_ref[...], dn, precision=lax.Precision.HIGHEST,
        preferred_element_type=jnp.float32)


def _project(node_embeddings, w1a, w1b, b1):
    return pl.pallas_call(
        _project_body,
        out_shape=[
            jax.ShapeDtypeStruct((N_NODES, HIDDEN), jnp.float32),
            jax.ShapeDtypeStruct((N_NODES, HIDDEN), jnp.float32),
        ],
    )(node_embeddings, w1a, w1b, b1)


_MESH = plsc.VectorSubcoreMesh(core_axis_name="c", subcore_axis_name="s")


@functools.partial(
    pl.kernel,
    mesh=_MESH,
    compiler_params=pltpu.CompilerParams(use_tc_tiling_on_sc=False,
                                         needs_layout_passes=False),
    out_type=jax.ShapeDtypeStruct((N_EDGES,), jnp.float32),
    scratch_types=[
        pltpu.VMEM((CHUNK,), jnp.int32),
        pltpu.VMEM((CHUNK,), jnp.int32),
        pltpu.VMEM((CHUNK,), jnp.int32),
        pltpu.VMEM((CHUNK,), jnp.int32),
        pltpu.VMEM((CHUNK, HIDDEN), jnp.float32),
        pltpu.VMEM((CHUNK, HIDDEN), jnp.float32),
        pltpu.VMEM((CHUNK, HIDDEN), jnp.float32),
        pltpu.VMEM((CHUNK, HIDDEN), jnp.float32),
        pltpu.VMEM((CHUNK,), jnp.float32),
        pltpu.VMEM((CHUNK,), jnp.float32),
        pltpu.VMEM((HIDDEN * LANES,), jnp.float32),
        pltpu.VMEM((LANES,), jnp.float32),
        pltpu.SemaphoreType.DMA,
        pltpu.SemaphoreType.DMA,
        pltpu.SemaphoreType.DMA,
        pltpu.SemaphoreType.DMA,
        pltpu.SemaphoreType.DMA,
        pltpu.SemaphoreType.DMA,
    ],
)
def _decode(ta_hbm, tb_hbm, src_hbm, dst_hbm, w2_hbm, b2_hbm, out_hbm,
            si0, di0, si1, di1, a0, b0, a1, b1v_, o0, o1, w2_v, b2_v,
            sem_i0, sem_i1, sem_g0, sem_g1, sem_o0, sem_o1):
    wid = lax.axis_index("s") * NC + lax.axis_index("c")
    base = wid * EDGES_PER_W
    pltpu.sync_copy(w2_hbm, w2_v)
    pltpu.sync_copy(b2_hbm, b2_v)
    lane = lax.iota(jnp.int32, LANES)

    bufs = [
        dict(si=si0, di=di0, a=a0, b=b0, o=o0,
             sem_i=sem_i0, sem_g=sem_g0, sem_o=sem_o0),
        dict(si=si1, di=di1, a=a1, b=b1v_, o=o1,
             sem_i=sem_i1, sem_g=sem_g1, sem_o=sem_o1),
    ]

    def off_of(c):
        return pl.multiple_of(base + c * CHUNK, 8)

    def idx_fetch(c, bf, start):
        off = off_of(c)
        for hbm, ref in ((src_hbm, bf["si"]), (dst_hbm, bf["di"])):
            cp = pltpu.make_async_copy(hbm.at[pl.ds(off, CHUNK)], ref,
                                       bf["sem_i"])
            cp.start() if start else cp.wait()

    def gathers(bf, start):
        for off, size in SUB_SPLITS:
            sl = pl.ds(off, size)
            for hbm, idx, ref in ((ta_hbm, bf["si"], bf["a"]),
                                  (tb_hbm, bf["di"], bf["b"])):
                cp = pltpu.make_async_copy(hbm.at[idx.at[sl]], ref.at[sl],
                                           bf["sem_g"])
                cp.start() if start else cp.wait()

    def out_copy(c, bf, start):
        off = off_of(c)
        cp = pltpu.make_async_copy(bf["o"], out_hbm.at[pl.ds(off, CHUNK)],
                                   bf["sem_o"])
        cp.start() if start else cp.wait()

    def compute(bf):
        def pair_groups(t, carry):
            g0 = t * 2
            rows0 = lane + g0 * LANES
            rows1 = rows0 + LANES
            acc0 = b2_v[...]
            acc1 = b2_v[...]
            for k in range(HIDDEN):
                cols = (lane + k) & (HIDDEN - 1)
                wv = w2_v[pl.ds(k * LANES, LANES)]
                av0 = plsc.load_gather(bf["a"], [rows0, cols])
                bv0 = plsc.load_gather(bf["b"], [rows0, cols])
                acc0 = acc0 + jnp.maximum(av0 + bv0, 0.0) * wv
                av1 = plsc.load_gather(bf["a"], [rows1, cols])
                bv1 = plsc.load_gather(bf["b"], [rows1, cols])
                acc1 = acc1 + jnp.maximum(av1 + bv1, 0.0) * wv
            off = pl.multiple_of(g0 * LANES, 8)
            bf["o"][pl.ds(off, LANES)] = acc0
            bf["o"][pl.ds(off + LANES, LANES)] = acc1
            return carry

        def single_group(g, carry):
            rows = lane + g * LANES
            acc = b2_v[...]
            for k in range(HIDDEN):
                cols = (lane + k) & (HIDDEN - 1)
                wv = w2_v[pl.ds(k * LANES, LANES)]
                av = plsc.load_gather(bf["a"], [rows, cols])
                bv = plsc.load_gather(bf["b"], [rows, cols])
                acc = acc + jnp.maximum(av + bv, 0.0) * wv
            bf["o"][pl.ds(pl.multiple_of(g * LANES, 8), LANES)] = acc
            return carry

        lax.fori_loop(0, GROUPS // 2, pair_groups, 0)
        single_group(GROUPS - 1, 0)

    idx_fetch(0, bufs[0], True)
    idx_fetch(0, bufs[0], False)
    gathers(bufs[0], True)
    idx_fetch(1, bufs[1], True)

    def half(c, par):
        cur, nxt = bufs[par], bufs[1 - par]

        gathers(cur, False)

        @pl.when(c + 1 < N_CHUNKS)
        def _():
            idx_fetch(c + 1, nxt, False)
            gathers(nxt, True)

        @pl.when(c + 2 < N_CHUNKS)
        def _():
            idx_fetch(c + 2, cur, True)

        @pl.when(c >= 2)
        def _():
            out_copy(c - 2, cur, False)

        compute(cur)
        out_copy(c, cur, True)

    def pair_body(t, carry):
        c = t * 2
        half(c, 0)

        @pl.when(c + 1 < N_CHUNKS)
        def _():
            half(c + 1, 1)
        return carry

    lax.fori_loop(0, (N_CHUNKS + 1) // 2, pair_body, 0)

    out_copy(N_CHUNKS - 2, bufs[(N_CHUNKS - 2) % 2], False)
    out_copy(N_CHUNKS - 1, bufs[(N_CHUNKS - 1) % 2], False)


def kernel(node_embeddings, edge_index, W1, b1, W2, b2):
    ei = edge_index.astype(jnp.int32)
    ta, tb = _project(node_embeddings, W1[:IN_CHANNELS], W1[IN_CHANNELS:],
                      b1.reshape(1, HIDDEN))
    kk = jnp.arange(HIDDEN, dtype=jnp.int32)[:, None]
    ll = jnp.arange(LANES, dtype=jnp.int32)[None, :]
    w2rot = W2.reshape(HIDDEN)[(kk + ll) % HIDDEN].reshape(HIDDEN * LANES)
    b2v = jnp.broadcast_to(b2.reshape(()), (LANES,))
    out = _decode(ta, tb, ei[0], ei[1], w2rot, b2v)
    return out.reshape(N_EDGES, 1)


# speedup vs baseline: 1.0292x; 1.0292x over previous
import functools

import jax
import jax.numpy as jnp
from jax import lax
from jax.experimental import pallas as pl
from jax.experimental.pallas import tpu as pltpu
from jax.experimental.pallas import tpu_sc as plsc

IN_CHANNELS = 128
HIDDEN = 64
N_NODES = 10000
N_EDGES = 320000

NC = 2
NS = 16
LANES = 16
NW = NC * NS
EDGES_PER_W = N_EDGES // NW
CHUNK = 400
N_CHUNKS = EDGES_PER_W // CHUNK
GROUPS = CHUNK // LANES
SUB_SPLITS = ((0, 128), (128, 128), (256, 128), (384, 16))


def _project_body(e_ref, w1a_ref, w1b_ref, b1_ref, ta_ref, tb_ref):
    e = e_ref[...]
    dn = (((1,), (0,)), ((), ()))
    ta_ref[...] = lax.dot_general(
        e, w1a_ref[...], dn, precision=lax.Precision.HIGHEST,
        preferred_element_type=jnp.float32) + b1_ref[...]
    tb_ref[...] = lax.dot_general(
        e, w1b_ref[...], dn, precision=lax.Precision.HIGHEST,
        preferred_element_type=jnp.float32)


def _project(node_embeddings, w1a, w1b, b1):
    return pl.pallas_call(
        _project_body,
        out_shape=[
            jax.ShapeDtypeStruct((N_NODES, HIDDEN), jnp.float32),
            jax.ShapeDtypeStruct((N_NODES, HIDDEN), jnp.float32),
        ],
    )(node_embeddings, w1a, w1b, b1)


_MESH = plsc.VectorSubcoreMesh(core_axis_name="c", subcore_axis_name="s")


@functools.partial(
    pl.kernel,
    mesh=_MESH,
    compiler_params=pltpu.CompilerParams(use_tc_tiling_on_sc=False,
                                         needs_layout_passes=False),
    out_type=jax.ShapeDtypeStruct((N_EDGES,), jnp.float32),
    scratch_types=[
        pltpu.VMEM((CHUNK,), jnp.int32),
        pltpu.VMEM((CHUNK,), jnp.int32),
        pltpu.VMEM((CHUNK,), jnp.int32),
        pltpu.VMEM((CHUNK,), jnp.int32),
        pltpu.VMEM((CHUNK, HIDDEN), jnp.float32),
        pltpu.VMEM((CHUNK, HIDDEN), jnp.float32),
        pltpu.VMEM((CHUNK, HIDDEN), jnp.float32),
        pltpu.VMEM((CHUNK, HIDDEN), jnp.float32),
        pltpu.VMEM((CHUNK,), jnp.float32),
        pltpu.VMEM((CHUNK,), jnp.float32),
        pltpu.VMEM((HIDDEN * LANES,), jnp.float32),
        pltpu.VMEM((LANES,), jnp.float32),
        pltpu.SemaphoreType.DMA,
        pltpu.SemaphoreType.DMA,
        pltpu.SemaphoreType.DMA,
        pltpu.SemaphoreType.DMA,
        pltpu.SemaphoreType.DMA,
        pltpu.SemaphoreType.DMA,
    ],
)
def _decode(ta_hbm, tb_hbm, src_hbm, dst_hbm, w2_hbm, b2_hbm, out_hbm,
            si0, di0, si1, di1, a0, b0, a1, b1v_, o0, o1, w2_v, b2_v,
            sem_i0, sem_i1, sem_g0, sem_g1, sem_o0, sem_o1):
    wid = lax.axis_index("s") * NC + lax.axis_index("c")
    base = wid * EDGES_PER_W
    pltpu.sync_copy(w2_hbm, w2_v)
    pltpu.sync_copy(b2_hbm, b2_v)
    lane = lax.iota(jnp.int32, LANES)

    bufs = [
        dict(si=si0, di=di0, a=a0, b=b0, o=o0,
             sem_i=sem_i0, sem_g=sem_g0, sem_o=sem_o0),
        dict(si=si1, di=di1, a=a1, b=b1v_, o=o1,
             sem_i=sem_i1, sem_g=sem_g1, sem_o=sem_o1),
    ]

    def off_of(c):
        return pl.multiple_of(base + c * CHUNK, 8)

    def idx_fetch(c, bf, start):
        off = off_of(c)
        for hbm, ref in ((src_hbm, bf["si"]), (dst_hbm, bf["di"])):
            cp = pltpu.make_async_copy(hbm.at[pl.ds(off, CHUNK)], ref,
                                       bf["sem_i"])
            cp.start() if start else cp.wait()

    def gathers(bf, start):
        for off, size in SUB_SPLITS:
            sl = pl.ds(off, size)
            for hbm, idx, ref in ((ta_hbm, bf["si"], bf["a"]),
                                  (tb_hbm, bf["di"], bf["b"])):
                cp = pltpu.make_async_copy(hbm.at[idx.at[sl]], ref.at[sl],
                                           bf["sem_g"])
                cp.start() if start else cp.wait()

    def out_copy(c, bf, start):
        off = off_of(c)
        cp = pltpu.make_async_copy(bf["o"], out_hbm.at[pl.ds(off, CHUNK)],
                                   bf["sem_o"])
        cp.start() if start else cp.wait()

    def compute(bf):
        def group_body(g, carry):
            rows = lane + g * LANES
            acc = b2_v[...]
            for k in range(HIDDEN):
                cols = (lane + k) & (HIDDEN - 1)
                wv = w2_v[pl.ds(k * LANES, LANES)]
                av = plsc.load_gather(bf["a"], [rows, cols])
                bv = plsc.load_gather(bf["b"], [rows, cols])
                acc = acc + jnp.maximum(av + bv, 0.0) * wv
            bf["o"][pl.ds(pl.multiple_of(g * LANES, 8), LANES)] = acc
            return carry

        lax.fori_loop(0, GROUPS, group_body, 0)

    idx_fetch(0, bufs[0], True)
    idx_fetch(0, bufs[0], False)
    gathers(bufs[0], True)
    idx_fetch(1, bufs[1], True)

    def half(c, par):
        cur, nxt = bufs[par], bufs[1 - par]

        gathers(cur, False)

        @pl.when(c + 1 < N_CHUNKS)
        def _():
            idx_fetch(c + 1, nxt, False)
            gathers(nxt, True)

        @pl.when(c + 2 < N_CHUNKS)
        def _():
            idx_fetch(c + 2, cur, True)

        @pl.when(c >= 2)
        def _():
            out_copy(c - 2, cur, False)

        compute(cur)
        out_copy(c, cur, True)

    def pair_body(t, carry):
        c = t * 2
        half(c, 0)

        @pl.when(c + 1 < N_CHUNKS)
        def _():
            half(c + 1, 1)
        return carry

    lax.fori_loop(0, (N_CHUNKS + 1) // 2, pair_body, 0)

    out_copy(N_CHUNKS - 2, bufs[(N_CHUNKS - 2) % 2], False)
    out_copy(N_CHUNKS - 1, bufs[(N_CHUNKS - 1) % 2], False)


def kernel(node_embeddings, edge_index, W1, b1, W2, b2):
    ei = edge_index.astype(jnp.int32)
    ta, tb = _project(node_embeddings, W1[:IN_CHANNELS], W1[IN_CHANNELS:],
                      b1.reshape(1, HIDDEN))
    kk = jnp.arange(HIDDEN, dtype=jnp.int32)[:, None]
    ll = jnp.arange(LANES, dtype=jnp.int32)[None, :]
    w2rot = W2.reshape(HIDDEN)[(kk + ll) % HIDDEN].reshape(HIDDEN * LANES)
    b2v = jnp.broadcast_to(b2.reshape(()), (LANES,))
    out = _decode(ta, tb, ei[0], ei[1], w2rot, b2v)
    return out.reshape(N_EDGES, 1)
